# Initial kernel scaffold; baseline (speedup 1.0000x reference)
import functools

import jax
import jax.numpy as jnp
from jax import lax
from jax.experimental import pallas as pl
from jax.experimental.pallas import tpu as pltpu
from jax.experimental.pallas import tpu_sc as plsc

U = 25000
I = 25000
N = U + I
E = 800000
D = 64

HALF = N // 2
ACC_ROWS = 25600
GARBAGE = ACC_ROWS - 8
C = 80
EDGES_PER_TILE = E // 16
CHUNKS = EDGES_PER_TILE // C

_mesh = plsc.VectorSubcoreMesh(core_axis_name="c", subcore_axis_name="s")


@functools.partial(
    pl.kernel,
    mesh=_mesh,
    out_type=jax.ShapeDtypeStruct((N, D), jnp.float32),
    scratch_types=[
        pltpu.VMEM((C,), jnp.int32),
        pltpu.VMEM((C,), jnp.int32),
        pltpu.VMEM((C,), jnp.float32),
        pltpu.VMEM((1, C), jnp.int32),
        pltpu.VMEM((C, D), jnp.float32),
        pltpu.VMEM((C, D), jnp.float32),
        pltpu.VMEM_SHARED((ACC_ROWS, D), jnp.float32),
        pltpu.SemaphoreType.DMA,
    ],
)
def _layer(ego, src_h, dst_h, val_h, out, src_i, dst_i, val_f, dl2, rows,
           zrow, acc, sem):
    cid = lax.axis_index("c")
    sid = lax.axis_index("s")
    base = cid * HALF

    zero16 = jnp.zeros((16,), jnp.float32)
    for r in range(C):
        for j in range(D // 16):
            zrow[r, pl.ds(j * 16, 16)] = zero16

    def zero_body(r, _):
        off = pl.multiple_of(sid * (ACC_ROWS // 16) + r * C, 8)
        pltpu.sync_copy(zrow, acc.at[pl.ds(off, C)])
        return _

    lax.fori_loop(0, ACC_ROWS // 16 // C, zero_body, None)
    plsc.subcore_barrier()

    def chunk_body(g, _):
        off = pl.multiple_of(sid * EDGES_PER_TILE + g * C, 8)
        pltpu.sync_copy(src_h.at[pl.ds(off, C)], src_i)
        pltpu.sync_copy(dst_h.at[pl.ds(off, C)], dst_i)
        pltpu.sync_copy(val_h.at[pl.ds(off, C)], val_f)
        pltpu.async_copy(ego.at[src_i], rows, sem).wait()
        for v in range(C // 16):
            d = dst_i[pl.ds(v * 16, 16)]
            dl = d - base
            ok = (dl >= 0) & (dl < HALF)
            dl2[0, pl.ds(v * 16, 16)] = jnp.where(ok, dl, GARBAGE)
        for e in range(C):
            vb = plsc.load_gather(val_f, [jnp.full((16,), e, jnp.int32)])
            for j in range(D // 16):
                sl = pl.ds(j * 16, 16)
                rows[e, sl] = rows[e, sl] * vb
        pltpu.sync_copy(rows, acc.at[dl2.at[0]], add=True)
        return _

    lax.fori_loop(0, CHUNKS, chunk_body, None)
    plsc.subcore_barrier()

    rows_per_tile = ACC_ROWS // 16
    last_rows = HALF - 15 * rows_per_tile

    @pl.when(sid < 15)
    def _():
        off = pl.multiple_of(sid * rows_per_tile, 8)
        pltpu.sync_copy(acc.at[pl.ds(off, rows_per_tile)],
                        out.at[pl.ds(base + off, rows_per_tile)])

    @pl.when(sid == 15)
    def _():
        off = 15 * rows_per_tile
        pltpu.sync_copy(acc.at[pl.ds(off, last_rows)],
                        out.at[pl.ds(base + off, last_rows)])


def _mean_body(a, b, c, d, o):
    o[...] = (a[...] + b[...] + c[...] + d[...]) * 0.25


_mean4 = pl.pallas_call(
    _mean_body,
    grid=(50,),
    in_specs=[pl.BlockSpec((1000, D), lambda i: (i, 0))] * 4,
    out_specs=pl.BlockSpec((1000, D), lambda i: (i, 0)),
    out_shape=jax.ShapeDtypeStruct((N, D), jnp.float32),
)


def kernel(edge_index, edge_values, user_emb, item_emb):
    ego0 = jnp.concatenate([user_emb, item_emb], axis=0)
    src = edge_index[0]
    dst = edge_index[1]
    e1 = _layer(ego0, src, dst, edge_values)
    e2 = _layer(e1, src, dst, edge_values)
    e3 = _layer(e2, src, dst, edge_values)
    final = _mean4(ego0, e1, e2, e3)
    return final[:U], final[U:]


# speedup vs baseline: 2.3399x; 2.3399x over previous
import functools

import jax
import jax.numpy as jnp
from jax import lax
from jax.experimental import pallas as pl
from jax.experimental.pallas import tpu as pltpu
from jax.experimental.pallas import tpu_sc as plsc

U = 25000
I = 25000
N = U + I
E = 800000
D = 64

HALF = N // 2
ACC_ROWS = 25600
GARBAGE = ACC_ROWS - 8
C = 80
EDGES_PER_TILE = E // 16
CHUNKS = EDGES_PER_TILE // C

_mesh = plsc.VectorSubcoreMesh(core_axis_name="c", subcore_axis_name="s")

_GATHER_DN = lax.GatherDimensionNumbers(
    offset_dims=(), collapsed_slice_dims=(0,), start_index_map=(0,))


def _bcast_lane(v, l):
    idx = jnp.full((16, 1), l, jnp.int32)
    return lax.gather(v, idx, _GATHER_DN, (1,),
                      mode=lax.GatherScatterMode.PROMISE_IN_BOUNDS)


@functools.partial(
    pl.kernel,
    mesh=_mesh,
    out_type=jax.ShapeDtypeStruct((N, D), jnp.float32),
    scratch_types=[
        pltpu.VMEM((C,), jnp.int32),
        pltpu.VMEM((C,), jnp.int32),
        pltpu.VMEM((C,), jnp.float32),
        pltpu.VMEM((1, C), jnp.int32),
        pltpu.VMEM((C, D), jnp.float32),
        pltpu.VMEM((C, D), jnp.float32),
        pltpu.VMEM_SHARED((ACC_ROWS, D), jnp.float32),
        pltpu.SemaphoreType.DMA,
    ],
    compiler_params=pltpu.CompilerParams(use_tc_tiling_on_sc=False),
)
def _layer(ego, src_h, dst_h, val_h, out, src_i, dst_i, val_f, dl2, rows,
           zrow, acc, sem):
    cid = lax.axis_index("c")
    sid = lax.axis_index("s")
    base = cid * HALF

    zero16 = jnp.zeros((16,), jnp.float32)
    for r in range(C):
        for j in range(D // 16):
            zrow[r, pl.ds(j * 16, 16)] = zero16

    def zero_body(r, _):
        off = pl.multiple_of(sid * (ACC_ROWS // 16) + r * C, 8)
        pltpu.sync_copy(zrow, acc.at[pl.ds(off, C)])
        return _

    lax.fori_loop(0, ACC_ROWS // 16 // C, zero_body, None)
    plsc.subcore_barrier()

    def chunk_body(g, _):
        off = pl.multiple_of(sid * EDGES_PER_TILE + g * C, 8)
        pltpu.sync_copy(src_h.at[pl.ds(off, C)], src_i)
        pltpu.sync_copy(dst_h.at[pl.ds(off, C)], dst_i)
        pltpu.sync_copy(val_h.at[pl.ds(off, C)], val_f)
        pltpu.async_copy(ego.at[src_i], rows, sem).wait()
        for v in range(C // 16):
            d = dst_i[pl.ds(v * 16, 16)]
            dl = d - base
            ok = (dl >= 0) & (dl < HALF)
            dl2[0, pl.ds(v * 16, 16)] = jnp.where(ok, dl, GARBAGE)
        for ve in range(C // 16):
            vbs = val_f[pl.ds(ve * 16, 16)]
            for l in range(16):
                e = ve * 16 + l
                vb = _bcast_lane(vbs, l)
                for j in range(D // 16):
                    sl = pl.ds(j * 16, 16)
                    rows[e, sl] = rows[e, sl] * vb
        pltpu.sync_copy(rows, acc.at[dl2.at[0]], add=True)
        return _

    lax.fori_loop(0, CHUNKS, chunk_body, None)
    plsc.subcore_barrier()

    rows_per_tile = ACC_ROWS // 16
    last_rows = HALF - 15 * rows_per_tile

    @pl.when(sid < 15)
    def _():
        off = pl.multiple_of(sid * rows_per_tile, 8)
        pltpu.sync_copy(acc.at[pl.ds(off, rows_per_tile)],
                        out.at[pl.ds(base + off, rows_per_tile)])

    @pl.when(sid == 15)
    def _():
        off = 15 * rows_per_tile
        pltpu.sync_copy(acc.at[pl.ds(off, last_rows)],
                        out.at[pl.ds(base + off, last_rows)])


def _mean_body(a, b, c, d, o):
    o[...] = (a[...] + b[...] + c[...] + d[...]) * 0.25


_mean4 = pl.pallas_call(
    _mean_body,
    grid=(50,),
    in_specs=[pl.BlockSpec((1000, D), lambda i: (i, 0))] * 4,
    out_specs=pl.BlockSpec((1000, D), lambda i: (i, 0)),
    out_shape=jax.ShapeDtypeStruct((N, D), jnp.float32),
)


def kernel(edge_index, edge_values, user_emb, item_emb):
    ego0 = jnp.concatenate([user_emb, item_emb], axis=0)
    src = edge_index[0]
    dst = edge_index[1]
    e1 = _layer(ego0, src, dst, edge_values)
    e2 = _layer(e1, src, dst, edge_values)
    e3 = _layer(e2, src, dst, edge_values)
    final = _mean4(ego0, e1, e2, e3)
    return final[:U], final[U:]


# speedup vs baseline: 6.1565x; 2.6311x over previous
import functools

import jax
import jax.numpy as jnp
from jax import lax
from jax.experimental import pallas as pl
from jax.experimental.pallas import tpu as pltpu
from jax.experimental.pallas import tpu_sc as plsc

U = 25000
I = 25000
N = U + I
E = 800000
D = 64

HALF = N // 2
ACC_ROWS = 25600
GARBAGE = ACC_ROWS - 8
C = 80
EDGES_PER_TILE = E // 16
B = 2000
BLOCKS = EDGES_PER_TILE // B
BCHUNKS = B // C

_mesh = plsc.VectorSubcoreMesh(core_axis_name="c", subcore_axis_name="s")

_GATHER_DN = lax.GatherDimensionNumbers(
    offset_dims=(), collapsed_slice_dims=(0,), start_index_map=(0,))


def _bcast_lane(v, l):
    idx = jnp.full((16, 1), l, jnp.int32)
    return lax.gather(v, idx, _GATHER_DN, (1,),
                      mode=lax.GatherScatterMode.PROMISE_IN_BOUNDS)


@functools.partial(
    pl.kernel,
    mesh=_mesh,
    out_type=jax.ShapeDtypeStruct((N, D), jnp.float32),
    scratch_types=[
        pltpu.VMEM((B,), jnp.int32),
        pltpu.VMEM((B,), jnp.int32),
        pltpu.VMEM((B,), jnp.float32),
        pltpu.VMEM((2, C), jnp.int32),
        pltpu.VMEM((2, C, D), jnp.float32),
        pltpu.VMEM_SHARED((ACC_ROWS, D), jnp.float32),
        pltpu.SemaphoreType.DMA,
        pltpu.SemaphoreType.DMA,
        pltpu.SemaphoreType.DMA,
        pltpu.SemaphoreType.DMA,
    ],
    compiler_params=pltpu.CompilerParams(use_tc_tiling_on_sc=False),
)
def _layer(ego, src_h, dst_h, val_h, out, src_i, dst_i, val_f, dl2, rows,
           acc, gsem0, gsem1, ssem0, ssem1):
    cid = lax.axis_index("c")
    sid = lax.axis_index("s")
    base = cid * HALF
    gsem = (gsem0, gsem1)
    ssem = (ssem0, ssem1)

    zero16 = jnp.zeros((16,), jnp.float32)
    for r in range(C):
        for j in range(D // 16):
            rows[0, r, pl.ds(j * 16, 16)] = zero16

    def zero_body(r, _):
        off = pl.multiple_of(sid * (ACC_ROWS // 16) + r * C, 8)
        pltpu.sync_copy(rows.at[0], acc.at[pl.ds(off, C)])
        return _

    lax.fori_loop(0, ACC_ROWS // 16 // C, zero_body, None)
    plsc.subcore_barrier()

    def issue_gather(g, p):
        goff = pl.multiple_of(g * C, 8)
        pltpu.async_copy(ego.at[src_i.at[pl.ds(goff, C)]],
                         rows.at[p], gsem[p])

    def stage(g, p, first):
        coff = pl.multiple_of(g * C, 8)
        if not first:
            pltpu.make_async_copy(rows.at[1 - p], acc.at[dl2.at[1 - p]],
                                  ssem[1 - p]).wait()

        @pl.when(g + 1 < BCHUNKS)
        def _():
            issue_gather(g + 1, 1 - p)

        for v in range(C // 16):
            d = dst_i[pl.ds(coff + v * 16, 16)]
            dl = d - base
            ok = (dl >= 0) & (dl < HALF)
            dl2[p, pl.ds(v * 16, 16)] = jnp.where(ok, dl, GARBAGE)

        pltpu.make_async_copy(ego.at[src_i.at[pl.ds(coff, C)]],
                              rows.at[p], gsem[p]).wait()

        for ve in range(C // 16):
            vbs = val_f[pl.ds(coff + ve * 16, 16)]
            for l in range(16):
                e = ve * 16 + l
                vb = _bcast_lane(vbs, l)
                for j in range(D // 16):
                    sl = pl.ds(j * 16, 16)
                    rows[p, e, sl] = rows[p, e, sl] * vb

        pltpu.async_copy(rows.at[p], acc.at[dl2.at[p]], ssem[p], add=True)

    def block_body(b, _):
        boff = pl.multiple_of(sid * EDGES_PER_TILE + b * B, 8)
        pltpu.sync_copy(src_h.at[pl.ds(boff, B)], src_i)
        pltpu.sync_copy(dst_h.at[pl.ds(boff, B)], dst_i)
        pltpu.sync_copy(val_h.at[pl.ds(boff, B)], val_f)

        issue_gather(0, 0)
        stage(0, 0, first=True)

        def pair_body(g2, _):
            stage(2 * g2 + 1, 1, first=False)
            stage(2 * g2 + 2, 0, first=False)
            return _

        lax.fori_loop(0, (BCHUNKS - 1) // 2, pair_body, None)
        pltpu.make_async_copy(rows.at[0], acc.at[dl2.at[0]], ssem[0]).wait()
        return _

    lax.fori_loop(0, BLOCKS, block_body, None)
    plsc.subcore_barrier()

    rows_per_tile = ACC_ROWS // 16
    last_rows = HALF - 15 * rows_per_tile

    @pl.when(sid < 15)
    def _():
        off = pl.multiple_of(sid * rows_per_tile, 8)
        pltpu.sync_copy(acc.at[pl.ds(off, rows_per_tile)],
                        out.at[pl.ds(base + off, rows_per_tile)])

    @pl.when(sid == 15)
    def _():
        off = 15 * rows_per_tile
        pltpu.sync_copy(acc.at[pl.ds(off, last_rows)],
                        out.at[pl.ds(base + off, last_rows)])


def _mean_body(a, b, c, d, o):
    o[...] = (a[...] + b[...] + c[...] + d[...]) * 0.25


_mean4 = pl.pallas_call(
    _mean_body,
    grid=(50,),
    in_specs=[pl.BlockSpec((1000, D), lambda i: (i, 0))] * 4,
    out_specs=pl.BlockSpec((1000, D), lambda i: (i, 0)),
    out_shape=jax.ShapeDtypeStruct((N, D), jnp.float32),
)


def kernel(edge_index, edge_values, user_emb, item_emb):
    ego0 = jnp.concatenate([user_emb, item_emb], axis=0)
    src = edge_index[0]
    dst = edge_index[1]
    e1 = _layer(ego0, src, dst, edge_values)
    e2 = _layer(e1, src, dst, edge_values)
    e3 = _layer(e2, src, dst, edge_values)
    final = _mean4(ego0, e1, e2, e3)
    return final[:U], final[U:]


# speedup vs baseline: 7.8342x; 1.2725x over previous
import functools

import jax
import jax.numpy as jnp
from jax import lax
from jax.experimental import pallas as pl
from jax.experimental.pallas import tpu as pltpu
from jax.experimental.pallas import tpu_sc as plsc

U = 25000
I = 25000
N = U + I
E = 800000
D = 64
DH = D // 2

C = 80
EDGES_PER_TILE = E // 16
B = 2000
BLOCKS = EDGES_PER_TILE // B
BCHUNKS = B // C
ZROWS = 125
ROWS_PER_TILE = N // 16

_mesh = plsc.VectorSubcoreMesh(core_axis_name="c", subcore_axis_name="s")

_GATHER_DN = lax.GatherDimensionNumbers(
    offset_dims=(), collapsed_slice_dims=(0,), start_index_map=(0,))


def _bcast_lane(v, l):
    idx = jnp.full((16, 1), l, jnp.int32)
    return lax.gather(v, idx, _GATHER_DN, (1,),
                      mode=lax.GatherScatterMode.PROMISE_IN_BOUNDS)


@functools.partial(
    pl.kernel,
    mesh=_mesh,
    out_type=[jax.ShapeDtypeStruct((2, N, DH), jnp.float32)] * 3,
    scratch_types=[
        pltpu.VMEM((B,), jnp.int32),
        pltpu.VMEM((B,), jnp.int32),
        pltpu.VMEM((B,), jnp.float32),
        pltpu.VMEM((2, C), jnp.int32),
        pltpu.VMEM((2, C, DH), jnp.float32),
        pltpu.VMEM((ZROWS, DH), jnp.float32),
        pltpu.VMEM_SHARED((N, DH), jnp.float32),
        pltpu.SemaphoreType.DMA,
        pltpu.SemaphoreType.DMA,
        pltpu.SemaphoreType.DMA,
        pltpu.SemaphoreType.DMA,
    ],
    compiler_params=pltpu.CompilerParams(use_tc_tiling_on_sc=False),
)
def _gcn3(ego, src_h, dst_h, val_h, out1, out2, out3, src_i, dst_i, val_f,
          dl2, rows, zblk, acc, gsem0, gsem1, ssem0, ssem1):
    cid = lax.axis_index("c")
    sid = lax.axis_index("s")
    gsem = (gsem0, gsem1)
    ssem = (ssem0, ssem1)

    zero16 = jnp.zeros((16,), jnp.float32)
    for r in range(ZROWS):
        for j in range(DH // 16):
            zblk[r, pl.ds(j * 16, 16)] = zero16

    def run_layer(table, dest):
        my_tab = table.at[cid]
        my_out = dest.at[cid]

        def zero_body(r, _):
            off = sid * ROWS_PER_TILE + r * ZROWS
            pltpu.sync_copy(zblk, acc.at[pl.ds(off, ZROWS)])
            return _

        lax.fori_loop(0, ROWS_PER_TILE // ZROWS, zero_body, None)
        plsc.subcore_barrier()

        def issue_gather(g, p):
            goff = pl.multiple_of(g * C, 8)
            pltpu.async_copy(my_tab.at[src_i.at[pl.ds(goff, C)]],
                             rows.at[p], gsem[p])

        def stage(g, p, first):
            coff = pl.multiple_of(g * C, 8)
            if not first:
                pltpu.make_async_copy(rows.at[1 - p], acc.at[dl2.at[1 - p]],
                                      ssem[1 - p]).wait()

            @pl.when(g + 1 < BCHUNKS)
            def _():
                issue_gather(g + 1, 1 - p)

            for v in range(C // 16):
                dl2[p, pl.ds(v * 16, 16)] = dst_i[pl.ds(coff + v * 16, 16)]

            pltpu.make_async_copy(my_tab.at[src_i.at[pl.ds(coff, C)]],
                                  rows.at[p], gsem[p]).wait()

            def scale_body(ve, _):
                vbs = val_f[pl.ds(coff + ve * 16, 16)]
                for l in range(16):
                    for j in range(DH // 16):
                        sl = pl.ds(j * 16, 16)
                        rows[p, ve * 16 + l, sl] = (
                            rows[p, ve * 16 + l, sl] * _bcast_lane(vbs, l))
                return _

            lax.fori_loop(0, C // 16, scale_body, None)

            pltpu.async_copy(rows.at[p], acc.at[dl2.at[p]], ssem[p],
                             add=True)

        def block_body(b, _):
            boff = pl.multiple_of(sid * EDGES_PER_TILE + b * B, 8)
            pltpu.sync_copy(src_h.at[pl.ds(boff, B)], src_i)
            pltpu.sync_copy(dst_h.at[pl.ds(boff, B)], dst_i)
            pltpu.sync_copy(val_h.at[pl.ds(boff, B)], val_f)

            issue_gather(0, 0)
            stage(0, 0, first=True)

            def pair_body(g2, _):
                stage(2 * g2 + 1, 1, first=False)
                stage(2 * g2 + 2, 0, first=False)
                return _

            lax.fori_loop(0, (BCHUNKS - 1) // 2, pair_body, None)
            pltpu.make_async_copy(rows.at[0], acc.at[dl2.at[0]],
                                  ssem[0]).wait()
            return _

        lax.fori_loop(0, BLOCKS, block_body, None)
        plsc.subcore_barrier()

        off = sid * ROWS_PER_TILE
        pltpu.sync_copy(acc.at[pl.ds(off, ROWS_PER_TILE)],
                        my_out.at[pl.ds(off, ROWS_PER_TILE)])
        plsc.subcore_barrier()

    run_layer(ego, out1)
    run_layer(out1, out2)
    run_layer(out2, out3)


def _mean_body(a, b, c, d, o):
    for h in range(2):
        o[:, pl.ds(h * DH, DH)] = (
            a[h] + b[h] + c[h] + d[h]) * 0.25


_mean4 = pl.pallas_call(
    _mean_body,
    grid=(50,),
    in_specs=[pl.BlockSpec((2, 1000, DH), lambda i: (0, i, 0))] * 4,
    out_specs=pl.BlockSpec((1000, D), lambda i: (i, 0)),
    out_shape=jax.ShapeDtypeStruct((N, D), jnp.float32),
)


def kernel(edge_index, edge_values, user_emb, item_emb):
    ego0 = jnp.concatenate([user_emb, item_emb], axis=0)
    ego0c = jnp.stack([ego0[:, :DH], ego0[:, DH:]], axis=0)
    src = edge_index[0]
    dst = edge_index[1]
    e1, e2, e3 = _gcn3(ego0c, src, dst, edge_values)
    final = _mean4(ego0c, e1, e2, e3)
    return final[:U], final[U:]


# speedup vs baseline: 11.5220x; 1.4707x over previous
import functools

import jax
import jax.numpy as jnp
from jax import lax
from jax.experimental import pallas as pl
from jax.experimental.pallas import tpu as pltpu
from jax.experimental.pallas import tpu_sc as plsc

U = 25000
I = 25000
N = U + I
E = 800000
D = 64
DH = D // 2

C = 80
EDGES_PER_TILE = E // 16
B = 2000
BLOCKS = EDGES_PER_TILE // B
BCHUNKS = B // C
ZROWS = 125
ROWS_PER_TILE = N // 16

_mesh = plsc.VectorSubcoreMesh(core_axis_name="c", subcore_axis_name="s")

_GATHER_DN = lax.GatherDimensionNumbers(
    offset_dims=(), collapsed_slice_dims=(0,), start_index_map=(0,))


def _bcast_lane(v, l):
    idx = jnp.full((16, 1), l, jnp.int32)
    return lax.gather(v, idx, _GATHER_DN, (1,),
                      mode=lax.GatherScatterMode.PROMISE_IN_BOUNDS)


@functools.partial(
    pl.kernel,
    mesh=_mesh,
    out_type=[jax.ShapeDtypeStruct((2, N, DH), jnp.float32)] * 3,
    scratch_types=[
        pltpu.VMEM((B,), jnp.int32),
        pltpu.VMEM((B // C, C), jnp.int32),
        pltpu.VMEM((B,), jnp.float32),
        pltpu.VMEM((6, C, DH), jnp.float32),
        pltpu.VMEM((ZROWS, DH), jnp.float32),
        pltpu.VMEM_SHARED((N, DH), jnp.float32),
    ] + [pltpu.SemaphoreType.DMA] * 12,
    compiler_params=pltpu.CompilerParams(use_tc_tiling_on_sc=False),
)
def _gcn3(ego, src_h, dst_h, val_h, out1, out2, out3, src_i, dstb, val_f,
          rows, zblk, acc, g0, g1, g2, g3, g4, g5, s0, s1, s2, s3, s4, s5):
    cid = lax.axis_index("c")
    sid = lax.axis_index("s")
    gsem = (g0, g1, g2, g3, g4, g5)
    ssem = (s0, s1, s2, s3, s4, s5)

    zero16 = jnp.zeros((16,), jnp.float32)
    for r in range(ZROWS):
        for j in range(DH // 16):
            zblk[r, pl.ds(j * 16, 16)] = zero16

    def run_layer(table, dest):
        my_tab = table.at[cid]
        my_out = dest.at[cid]

        def zero_body(r, _):
            off = sid * ROWS_PER_TILE + r * ZROWS
            pltpu.sync_copy(zblk, acc.at[pl.ds(off, ZROWS)])
            return _

        lax.fori_loop(0, ROWS_PER_TILE // ZROWS, zero_body, None)
        plsc.subcore_barrier()

        def issue_gather(g, p):
            goff = pl.multiple_of(g * C, 8)
            pltpu.async_copy(my_tab.at[src_i.at[pl.ds(goff, C)]],
                             rows.at[p], gsem[p])

        def stage(g, p):
            coff = pl.multiple_of(g * C, 8)
            q = (p + 3) % 6

            @pl.when(g + 3 < BCHUNKS)
            def _():
                @pl.when(g >= 3)
                def _():
                    pltpu.make_async_copy(rows.at[q],
                                          acc.at[dstb.at[g - 3]],
                                          ssem[q]).wait()

                issue_gather(g + 3, q)

            pltpu.make_async_copy(my_tab.at[src_i.at[pl.ds(coff, C)]],
                                  rows.at[p], gsem[p]).wait()

            def scale_body(ve, _):
                vbs = val_f[pl.ds(coff + ve * 16, 16)]
                for l in range(16):
                    for j in range(DH // 16):
                        sl = pl.ds(j * 16, 16)
                        rows[p, ve * 16 + l, sl] = (
                            rows[p, ve * 16 + l, sl] * _bcast_lane(vbs, l))
                return _

            lax.fori_loop(0, C // 16, scale_body, None)

            pltpu.async_copy(rows.at[p], acc.at[dstb.at[g]], ssem[p],
                             add=True)

        def block_body(b, _):
            boff = pl.multiple_of(sid * EDGES_PER_TILE + b * B, 8)
            brow = sid * (EDGES_PER_TILE // C) + b * (B // C)
            pltpu.sync_copy(src_h.at[pl.ds(boff, B)], src_i)
            pltpu.sync_copy(dst_h.at[pl.ds(brow, B // C)], dstb)
            pltpu.sync_copy(val_h.at[pl.ds(boff, B)], val_f)

            issue_gather(0, 0)
            issue_gather(1, 1)
            issue_gather(2, 2)
            stage(0, 0)

            def six_body(k, _):
                for j in range(6):
                    stage(1 + 6 * k + j, (1 + j) % 6)
                return _

            lax.fori_loop(0, (BCHUNKS - 1) // 6, six_body, None)
            for i, bb in enumerate((1, 2, 3, 4, 5, 0)):
                pltpu.make_async_copy(
                    rows.at[bb], acc.at[dstb.at[BCHUNKS - 6 + i]],
                    ssem[bb]).wait()
            return _

        lax.fori_loop(0, BLOCKS, block_body, None)
        plsc.subcore_barrier()

        off = sid * ROWS_PER_TILE
        pltpu.sync_copy(acc.at[pl.ds(off, ROWS_PER_TILE)],
                        my_out.at[pl.ds(off, ROWS_PER_TILE)])
        plsc.subcore_barrier()

    run_layer(ego, out1)
    run_layer(out1, out2)
    run_layer(out2, out3)


def _mean_body(a, b, c, d, o):
    for h in range(2):
        o[:, pl.ds(h * DH, DH)] = (
            a[h] + b[h] + c[h] + d[h]) * 0.25


_mean4 = pl.pallas_call(
    _mean_body,
    grid=(50,),
    in_specs=[pl.BlockSpec((2, 1000, DH), lambda i: (0, i, 0))] * 4,
    out_specs=pl.BlockSpec((1000, D), lambda i: (i, 0)),
    out_shape=jax.ShapeDtypeStruct((N, D), jnp.float32),
)


def kernel(edge_index, edge_values, user_emb, item_emb):
    ego0 = jnp.concatenate([user_emb, item_emb], axis=0)
    ego0c = jnp.stack([ego0[:, :DH], ego0[:, DH:]], axis=0)
    src = edge_index[0]
    dst2d = edge_index[1].reshape(E // C, C)
    e1, e2, e3 = _gcn3(ego0c, src, dst2d, edge_values)
    final = _mean4(ego0c, e1, e2, e3)
    return final[:U], final[U:]
